# baseline (device time: 21881 ns/iter reference)
import jax
import jax.numpy as jnp
from jax import lax
from jax.experimental import pallas as pl
from jax.experimental.pallas import tpu as pltpu

_BLOCK_M = 256
_EPS = 1e-5


def kernel(x, dy, gamma):
    m, d = x.shape
    n_blocks = m // _BLOCK_M

    def body(x_ref, dy_ref, gamma_ref, out_ref, acc_ref, recv_ref,
             send_sem, recv_sem):
        i = pl.program_id(0)

        my_x = lax.axis_index("x")
        my_y = lax.axis_index("y")
        my_z = lax.axis_index("z")
        peer = (my_x, 1 - my_y, my_z)

        @pl.when(i == 0)
        def _():
            barrier_sem = pltpu.get_barrier_semaphore()
            pl.semaphore_signal(
                barrier_sem, inc=1, device_id=peer,
                device_id_type=pl.DeviceIdType.MESH,
            )

        xb = x_ref[:, :]
        dyb = dy_ref[:, :]
        s1 = jnp.sum(xb, axis=1, keepdims=True)
        s2 = jnp.sum(xb * xb, axis=1, keepdims=True)
        mu = s1 * (1.0 / d)
        var = s2 * (1.0 / d) - mu * mu
        xhat = (xb - mu) * lax.rsqrt(var + _EPS)
        dgamma = jnp.sum(dyb * xhat, axis=0, keepdims=True)
        dbeta = jnp.sum(dyb, axis=0, keepdims=True)

        @pl.when(i == 0)
        def _():
            acc_ref[0:1, :] = dgamma
            acc_ref[1:2, :] = dbeta

        @pl.when(i > 0)
        def _():
            acc_ref[0:1, :] = acc_ref[0:1, :] + dgamma
            acc_ref[1:2, :] = acc_ref[1:2, :] + dbeta

        @pl.when(i == n_blocks - 1)
        def _():
            barrier_sem = pltpu.get_barrier_semaphore()
            pl.semaphore_wait(barrier_sem, 1)

            rdma = pltpu.make_async_remote_copy(
                src_ref=acc_ref,
                dst_ref=recv_ref,
                send_sem=send_sem,
                recv_sem=recv_sem,
                device_id=peer,
                device_id_type=pl.DeviceIdType.MESH,
            )
            rdma.start()
            rdma.wait()
            out_ref[:, :] = acc_ref[:, :] + recv_ref[:, :]

    return pl.pallas_call(
        body,
        grid=(n_blocks,),
        out_shape=jax.ShapeDtypeStruct((2, d), jnp.float32),
        in_specs=[
            pl.BlockSpec((_BLOCK_M, d), lambda i: (i, 0)),
            pl.BlockSpec((_BLOCK_M, d), lambda i: (i, 0)),
            pl.BlockSpec(memory_space=pl.ANY),
        ],
        out_specs=pl.BlockSpec((2, d), lambda i: (0, 0)),
        scratch_shapes=[
            pltpu.VMEM((2, d), jnp.float32),
            pltpu.VMEM((2, d), jnp.float32),
            pltpu.SemaphoreType.DMA,
            pltpu.SemaphoreType.DMA,
        ],
        compiler_params=pltpu.CompilerParams(collective_id=0),
    )(x, dy, gamma)


# device time: 20461 ns/iter; 1.0694x vs baseline; 1.0694x over previous
import jax
import jax.numpy as jnp
from jax import lax
from jax.experimental import pallas as pl
from jax.experimental.pallas import tpu as pltpu

_BLOCK_M = 512
_EPS = 1e-5


def kernel(x, dy, gamma):
    m, d = x.shape
    n_blocks = m // _BLOCK_M

    def body(x_ref, dy_ref, gamma_ref, out_ref, acc_ref, recv_ref,
             send_sem, recv_sem):
        i = pl.program_id(0)

        my_x = lax.axis_index("x")
        my_y = lax.axis_index("y")
        my_z = lax.axis_index("z")
        peer = (my_x, 1 - my_y, my_z)

        @pl.when(i == 0)
        def _():
            barrier_sem = pltpu.get_barrier_semaphore()
            pl.semaphore_signal(
                barrier_sem, inc=1, device_id=peer,
                device_id_type=pl.DeviceIdType.MESH,
            )

        xb = x_ref[:, :]
        dyb = dy_ref[:, :]
        s1 = jnp.sum(xb, axis=1, keepdims=True)
        s2 = jnp.sum(xb * xb, axis=1, keepdims=True)
        mu = s1 * (1.0 / d)
        var = s2 * (1.0 / d) - mu * mu
        xhat = (xb - mu) * lax.rsqrt(var + _EPS)
        dgamma = jnp.sum(dyb * xhat, axis=0, keepdims=True)
        dbeta = jnp.sum(dyb, axis=0, keepdims=True)

        @pl.when(i == 0)
        def _():
            acc_ref[0:1, :] = dgamma
            acc_ref[1:2, :] = dbeta

        @pl.when(i > 0)
        def _():
            acc_ref[0:1, :] = acc_ref[0:1, :] + dgamma
            acc_ref[1:2, :] = acc_ref[1:2, :] + dbeta

        @pl.when(i == n_blocks - 1)
        def _():
            barrier_sem = pltpu.get_barrier_semaphore()
            pl.semaphore_wait(barrier_sem, 1)

            rdma = pltpu.make_async_remote_copy(
                src_ref=acc_ref,
                dst_ref=recv_ref,
                send_sem=send_sem,
                recv_sem=recv_sem,
                device_id=peer,
                device_id_type=pl.DeviceIdType.MESH,
            )
            rdma.start()
            rdma.wait()
            out_ref[:, :] = acc_ref[:, :] + recv_ref[:, :]

    return pl.pallas_call(
        body,
        grid=(n_blocks,),
        out_shape=jax.ShapeDtypeStruct((2, d), jnp.float32),
        in_specs=[
            pl.BlockSpec((_BLOCK_M, d), lambda i: (i, 0)),
            pl.BlockSpec((_BLOCK_M, d), lambda i: (i, 0)),
            pl.BlockSpec(memory_space=pl.ANY),
        ],
        out_specs=pl.BlockSpec((2, d), lambda i: (0, 0)),
        scratch_shapes=[
            pltpu.VMEM((2, d), jnp.float32),
            pltpu.VMEM((2, d), jnp.float32),
            pltpu.SemaphoreType.DMA,
            pltpu.SemaphoreType.DMA,
        ],
        compiler_params=pltpu.CompilerParams(collective_id=0),
    )(x, dy, gamma)
